# baseline (speedup 1.0000x reference)
import functools

import jax
import jax.numpy as jnp
from jax import lax
from jax.experimental import pallas as pl
from jax.experimental.pallas import tpu as pltpu
from jax.experimental.pallas import tpu_sc as plsc

_NC = 2
_NS = 16
_NW = _NC * _NS
_CHUNK = 256
_WB = 448
_D = 16
_K = 1


def _head_body(x_ref, w_ref, o_ref):
    res = jnp.dot(x_ref[...], w_ref[...], preferred_element_type=jnp.float32)
    o_ref[0] = res[:, :_D]
    o_ref[1] = res[:, _D:]


def _head(x, w, block_rows=2000):
    n, k = x.shape
    return pl.pallas_call(
        _head_body,
        grid=(n // block_rows,),
        in_specs=[
            pl.BlockSpec((block_rows, k), lambda i: (i, 0)),
            pl.BlockSpec((k, 2 * _D), lambda i: (0, 0)),
        ],
        out_specs=pl.BlockSpec((2, block_rows, _D), lambda i: (0, i, 0)),
        out_shape=jax.ShapeDtypeStruct((2, n, _D), jnp.float32),
    )(x, w)


def _mid_body(lo0_ref, lo1_ref, hi0_ref, hi1_ref, y0_ref, y1_ref,
              b1l_ref, b1h_ref, w2l_ref, w2h_ref, b2_ref, w3_ref, z_ref):
    h_lo = jnp.maximum(y0_ref[0] + lo0_ref[0] + lo1_ref[0] + b1l_ref[...], 0.0)
    h_hi = jnp.maximum(y1_ref[0] + hi0_ref[0] + hi1_ref[0] + b1h_ref[...], 0.0)
    t = (jnp.dot(h_lo, w2l_ref[...], preferred_element_type=jnp.float32)
         + jnp.dot(h_hi, w2h_ref[...], preferred_element_type=jnp.float32))
    h = jnp.maximum(t + b2_ref[...], 0.0)
    z_ref[...] = jnp.dot(h, w3_ref[...], preferred_element_type=jnp.float32)


def _mid(parts_lo, parts_hi, y2, b1, w2, b2, w3, block_rows=2000):
    n = y2.shape[1]
    m = w3.shape[1]
    part_spec0 = pl.BlockSpec((1, block_rows, _D), lambda i: (0, i, 0))
    part_spec1 = pl.BlockSpec((1, block_rows, _D), lambda i: (1, i, 0))
    return pl.pallas_call(
        _mid_body,
        grid=(n // block_rows,),
        in_specs=[
            part_spec0, part_spec1, part_spec0, part_spec1,
            part_spec0, part_spec1,
            pl.BlockSpec((1, _D), lambda i: (0, 0)),
            pl.BlockSpec((1, _D), lambda i: (0, 0)),
            pl.BlockSpec((_D, 2 * _D), lambda i: (0, 0)),
            pl.BlockSpec((_D, 2 * _D), lambda i: (0, 0)),
            pl.BlockSpec((1, 2 * _D), lambda i: (0, 0)),
            pl.BlockSpec((2 * _D, m), lambda i: (0, 0)),
        ],
        out_specs=pl.BlockSpec((block_rows, m), lambda i: (i, 0)),
        out_shape=jax.ShapeDtypeStruct((n, m), jnp.float32),
    )(parts_lo, parts_lo, parts_hi, parts_hi, y2, y2,
      b1[:, :_D], b1[:, _D:], w2[:_D, :], w2[_D:, :], b2, w3)


def _out_body(q0_ref, q1_ref, z_ref, b3_ref, w4_ref, b4_ref, o_ref):
    h = z_ref[...] + q0_ref[0] + q1_ref[0] + b3_ref[...]
    h = jnp.maximum(h, 0.0)
    o_ref[...] = jnp.dot(h, w4_ref[...],
                         preferred_element_type=jnp.float32) + b4_ref[...]


def _final(parts, z, b3, w4, b4, block_rows=2000):
    n, d = z.shape
    m = w4.shape[1]
    return pl.pallas_call(
        _out_body,
        grid=(n // block_rows,),
        in_specs=[
            pl.BlockSpec((1, block_rows, d), lambda i: (0, i, 0)),
            pl.BlockSpec((1, block_rows, d), lambda i: (1, i, 0)),
            pl.BlockSpec((block_rows, d), lambda i: (i, 0)),
            pl.BlockSpec((1, d), lambda i: (0, 0)),
            pl.BlockSpec((d, m), lambda i: (0, 0)),
            pl.BlockSpec((1, m), lambda i: (0, 0)),
        ],
        out_specs=pl.BlockSpec((block_rows, m), lambda i: (i, 0)),
        out_shape=jax.ShapeDtypeStruct((n, m), jnp.float32),
    )(parts, parts, z, b3, w4, b4)


@functools.lru_cache(maxsize=None)
def _make_agg(n_acc, rows_per_tile, n_wb, c1):
    mesh = plsc.VectorSubcoreMesh(core_axis_name="c", subcore_axis_name="s")

    @functools.partial(
        pl.kernel,
        out_type=jax.ShapeDtypeStruct((_NC, n_acc, _D), jnp.float32),
        mesh=mesh,
        compiler_params=pltpu.CompilerParams(use_tc_tiling_on_sc=False),
        scratch_types=[
            pltpu.VMEM((c1, _CHUNK), jnp.int32),
            pltpu.VMEM((c1, _CHUNK), jnp.int32),
            pltpu.VMEM((4, _K, _CHUNK, _D), jnp.float32),
            pltpu.VMEM((_WB, _D), jnp.float32),
            pltpu.VMEM_SHARED((n_acc, _D), jnp.float32),
            pltpu.SemaphoreType.DMA((4,)),
        ],
    )
    def agg(y_hbm, src_hbm, dst_hbm, out_hbm, src_v, dst_v, bufs_v, wb_v,
            acc_sh, gsem):
        c = lax.axis_index("c")
        s = lax.axis_index("s")
        wid = s * _NC + c
        base = s * rows_per_tile
        ng = c1 // _K

        zero16 = jnp.zeros((16,), jnp.float32)

        def zero_body(i, carry):
            wb_v[i, pl.ds(0, 16)] = zero16
            return carry

        lax.fori_loop(0, _WB, zero_body, 0)
        for k in range(n_wb):
            pltpu.sync_copy(wb_v, acc_sh.at[pl.ds(base + k * _WB, _WB)])

        pltpu.sync_copy(src_hbm.at[wid], src_v)
        pltpu.sync_copy(dst_hbm.at[wid], dst_v)
        plsc.subcore_barrier()

        def fire_gather(g, p):
            for b in range(_K):
                pltpu.async_copy(y_hbm.at[src_v.at[g * _K + b]],
                                 bufs_v.at[p, b], gsem.at[p])

        def drain_gather(g, p):
            for b in range(_K):
                pltpu.make_async_copy(y_hbm.at[src_v.at[g * _K + b]],
                                      bufs_v.at[p, b], gsem.at[p]).wait()

        def scat(g, p):
            for b in range(_K):
                pltpu.sync_copy(bufs_v.at[p, b],
                                acc_sh.at[dst_v.at[g * _K + b]], add=True)

        fire_gather(0, 0)
        fire_gather(1, 1)

        def block(g, carry):
            p = lax.rem(g, 4)
            p2 = lax.rem(g + 2, 4)
            drain_gather(g, p)

            @pl.when(g + 2 < ng)
            def _():
                fire_gather(g + 2, p2)

            scat(g, p)
            return carry

        lax.fori_loop(0, ng, block, 0)
        plsc.subcore_barrier()

        for k in range(n_wb):
            r0 = base + k * _WB
            pltpu.sync_copy(acc_sh.at[pl.ds(r0, _WB)],
                            out_hbm.at[c, pl.ds(r0, _WB)])

    return agg


def kernel(x, edge_index, w1, b1, w2, b2, w3, b3, w4, b4):
    n = x.shape[0]
    e = edge_index.shape[1]
    ei = edge_index.astype(jnp.int32)

    c1 = -(-e // (_NW * _CHUNK))
    c1 = -(-c1 // _K) * _K
    e_pad = _NW * c1 * _CHUNK
    rows_per_tile = -(-n // (_NS * _WB)) * _WB
    n_wb = rows_per_tile // _WB
    n_acc = _NS * rows_per_tile

    src = jnp.concatenate(
        [ei[0], jnp.zeros((e_pad - e,), jnp.int32)]).reshape(_NW, c1, _CHUNK)
    dst = jnp.concatenate(
        [ei[1], jnp.full((e_pad - e,), n, jnp.int32)]).reshape(_NW, c1, _CHUNK)

    agg = _make_agg(n_acc, rows_per_tile, n_wb, c1)

    y2 = _head(x, w1)
    parts_lo = agg(y2[0], src, dst)
    y_hi = y2[1] + parts_lo[0, 0, 0] * 0.0
    parts_hi = agg(y_hi, src, dst)
    z = _mid(parts_lo, parts_hi, y2, b1.reshape(1, -1), w2,
             b2.reshape(1, -1), w3)
    parts2 = agg(z, src, dst)
    return _final(parts2, z, b3.reshape(1, -1), w4, b4.reshape(1, -1))


# speedup vs baseline: 1.0115x; 1.0115x over previous
import functools

import jax
import jax.numpy as jnp
from jax import lax
from jax.experimental import pallas as pl
from jax.experimental.pallas import tpu as pltpu
from jax.experimental.pallas import tpu_sc as plsc

_NC = 2
_NS = 16
_NW = _NC * _NS
_CHUNK = 256
_WB = 448
_D = 16
_K = 1


def _head_body(x_ref, w_ref, o_ref):
    res = jnp.dot(x_ref[...], w_ref[...], preferred_element_type=jnp.float32)
    o_ref[0] = res[:, :_D]
    o_ref[1] = res[:, _D:]


def _head(x, w, block_rows=2000):
    n, k = x.shape
    return pl.pallas_call(
        _head_body,
        grid=(n // block_rows,),
        in_specs=[
            pl.BlockSpec((block_rows, k), lambda i: (i, 0)),
            pl.BlockSpec((k, 2 * _D), lambda i: (0, 0)),
        ],
        out_specs=pl.BlockSpec((2, block_rows, _D), lambda i: (0, i, 0)),
        out_shape=jax.ShapeDtypeStruct((2, n, _D), jnp.float32),
    )(x, w)


def _mid_body(lo0_ref, lo1_ref, hi0_ref, hi1_ref, y0_ref, y1_ref,
              b1l_ref, b1h_ref, w2l_ref, w2h_ref, b2_ref, w3_ref, z_ref):
    h_lo = jnp.maximum(y0_ref[0] + lo0_ref[0] + lo1_ref[0] + b1l_ref[...], 0.0)
    h_hi = jnp.maximum(y1_ref[0] + hi0_ref[0] + hi1_ref[0] + b1h_ref[...], 0.0)
    t = (jnp.dot(h_lo, w2l_ref[...], preferred_element_type=jnp.float32)
         + jnp.dot(h_hi, w2h_ref[...], preferred_element_type=jnp.float32))
    h = jnp.maximum(t + b2_ref[...], 0.0)
    z_ref[...] = jnp.dot(h, w3_ref[...], preferred_element_type=jnp.float32)


def _mid(parts_lo, parts_hi, y2, b1, w2, b2, w3, block_rows=2000):
    n = y2.shape[1]
    m = w3.shape[1]
    part_spec0 = pl.BlockSpec((1, block_rows, _D), lambda i: (0, i, 0))
    part_spec1 = pl.BlockSpec((1, block_rows, _D), lambda i: (1, i, 0))
    return pl.pallas_call(
        _mid_body,
        grid=(n // block_rows,),
        in_specs=[
            part_spec0, part_spec1, part_spec0, part_spec1,
            part_spec0, part_spec1,
            pl.BlockSpec((1, _D), lambda i: (0, 0)),
            pl.BlockSpec((1, _D), lambda i: (0, 0)),
            pl.BlockSpec((_D, 2 * _D), lambda i: (0, 0)),
            pl.BlockSpec((_D, 2 * _D), lambda i: (0, 0)),
            pl.BlockSpec((1, 2 * _D), lambda i: (0, 0)),
            pl.BlockSpec((2 * _D, m), lambda i: (0, 0)),
        ],
        out_specs=pl.BlockSpec((block_rows, m), lambda i: (i, 0)),
        out_shape=jax.ShapeDtypeStruct((n, m), jnp.float32),
    )(parts_lo, parts_lo, parts_hi, parts_hi, y2, y2,
      b1[:, :_D], b1[:, _D:], w2[:_D, :], w2[_D:, :], b2, w3)


def _out_body(q0_ref, q1_ref, z_ref, b3_ref, w4_ref, b4_ref, o_ref):
    h = z_ref[...] + q0_ref[0] + q1_ref[0] + b3_ref[...]
    h = jnp.maximum(h, 0.0)
    o_ref[...] = jnp.dot(h, w4_ref[...],
                         preferred_element_type=jnp.float32) + b4_ref[...]


def _final(parts, z, b3, w4, b4, block_rows=2000):
    n, d = z.shape
    m = w4.shape[1]
    return pl.pallas_call(
        _out_body,
        grid=(n // block_rows,),
        in_specs=[
            pl.BlockSpec((1, block_rows, d), lambda i: (0, i, 0)),
            pl.BlockSpec((1, block_rows, d), lambda i: (1, i, 0)),
            pl.BlockSpec((block_rows, d), lambda i: (i, 0)),
            pl.BlockSpec((1, d), lambda i: (0, 0)),
            pl.BlockSpec((d, m), lambda i: (0, 0)),
            pl.BlockSpec((1, m), lambda i: (0, 0)),
        ],
        out_specs=pl.BlockSpec((block_rows, m), lambda i: (i, 0)),
        out_shape=jax.ShapeDtypeStruct((n, m), jnp.float32),
    )(parts, parts, z, b3, w4, b4)


@functools.lru_cache(maxsize=None)
def _make_agg(n_acc, rows_per_tile, n_wb, c1):
    mesh = plsc.VectorSubcoreMesh(core_axis_name="c", subcore_axis_name="s")

    @functools.partial(
        pl.kernel,
        out_type=jax.ShapeDtypeStruct((_NC, n_acc, _D), jnp.float32),
        mesh=mesh,
        compiler_params=pltpu.CompilerParams(use_tc_tiling_on_sc=False),
        scratch_types=[
            pltpu.VMEM((c1, _CHUNK), jnp.int32),
            pltpu.VMEM((c1, _CHUNK), jnp.int32),
            pltpu.VMEM((4, _K, _CHUNK, _D), jnp.float32),
            pltpu.VMEM((_WB, _D), jnp.float32),
            pltpu.VMEM_SHARED((n_acc, _D), jnp.float32),
            pltpu.SemaphoreType.DMA((4,)),
            pltpu.SemaphoreType.DMA((4,)),
        ],
    )
    def agg(y_hbm, src_hbm, dst_hbm, out_hbm, src_v, dst_v, bufs_v, wb_v,
            acc_sh, gsem, ssem):
        c = lax.axis_index("c")
        s = lax.axis_index("s")
        wid = s * _NC + c
        base = s * rows_per_tile
        ng = c1 // _K

        zero16 = jnp.zeros((16,), jnp.float32)

        def zero_body(i, carry):
            wb_v[i, pl.ds(0, 16)] = zero16
            return carry

        lax.fori_loop(0, _WB, zero_body, 0)
        for k in range(n_wb):
            pltpu.sync_copy(wb_v, acc_sh.at[pl.ds(base + k * _WB, _WB)])

        pltpu.sync_copy(src_hbm.at[wid], src_v)
        pltpu.sync_copy(dst_hbm.at[wid], dst_v)
        plsc.subcore_barrier()

        def fire_gather(g, p):
            for b in range(_K):
                pltpu.async_copy(y_hbm.at[src_v.at[g * _K + b]],
                                 bufs_v.at[p, b], gsem.at[p])

        def drain_gather(g, p):
            for b in range(_K):
                pltpu.make_async_copy(y_hbm.at[src_v.at[g * _K + b]],
                                      bufs_v.at[p, b], gsem.at[p]).wait()

        def fire_scatter(g, p):
            for b in range(_K):
                pltpu.async_copy(bufs_v.at[p, b],
                                 acc_sh.at[dst_v.at[g * _K + b]],
                                 ssem.at[p], add=True)

        def drain_scatter(g, p):
            for b in range(_K):
                pltpu.make_async_copy(bufs_v.at[p, b],
                                      acc_sh.at[dst_v.at[g * _K + b]],
                                      ssem.at[p]).wait()

        fire_gather(0, 0)
        fire_gather(1, 1)

        def block(g, carry):
            p = lax.rem(g, 4)
            p2 = lax.rem(g + 2, 4)
            drain_gather(g, p)
            fire_scatter(g, p)

            @pl.when(g >= 2)
            def _():
                drain_scatter(g - 2, p2)

            @pl.when(g + 2 < ng)
            def _():
                fire_gather(g + 2, p2)

            return carry

        lax.fori_loop(0, ng, block, 0)
        drain_scatter(ng - 2, (ng - 2) % 4)
        drain_scatter(ng - 1, (ng - 1) % 4)
        plsc.subcore_barrier()

        for k in range(n_wb):
            r0 = base + k * _WB
            pltpu.sync_copy(acc_sh.at[pl.ds(r0, _WB)],
                            out_hbm.at[c, pl.ds(r0, _WB)])

    return agg


def kernel(x, edge_index, w1, b1, w2, b2, w3, b3, w4, b4):
    n = x.shape[0]
    e = edge_index.shape[1]
    ei = edge_index.astype(jnp.int32)

    c1 = -(-e // (_NW * _CHUNK))
    c1 = -(-c1 // _K) * _K
    e_pad = _NW * c1 * _CHUNK
    rows_per_tile = -(-n // (_NS * _WB)) * _WB
    n_wb = rows_per_tile // _WB
    n_acc = _NS * rows_per_tile

    src = jnp.concatenate(
        [ei[0], jnp.zeros((e_pad - e,), jnp.int32)]).reshape(_NW, c1, _CHUNK)
    dst = jnp.concatenate(
        [ei[1], jnp.full((e_pad - e,), n, jnp.int32)]).reshape(_NW, c1, _CHUNK)

    agg = _make_agg(n_acc, rows_per_tile, n_wb, c1)

    y2 = _head(x, w1)
    parts_lo = agg(y2[0], src, dst)
    y_hi = y2[1] + parts_lo[0, 0, 0] * 0.0
    parts_hi = agg(y_hi, src, dst)
    z = _mid(parts_lo, parts_hi, y2, b1.reshape(1, -1), w2,
             b2.reshape(1, -1), w3)
    parts2 = agg(z, src, dst)
    return _final(parts2, z, b3.reshape(1, -1), w4, b4.reshape(1, -1))


# speedup vs baseline: 1.0880x; 1.0757x over previous
import functools

import jax
import jax.numpy as jnp
from jax import lax
from jax.experimental import pallas as pl
from jax.experimental.pallas import tpu as pltpu
from jax.experimental.pallas import tpu_sc as plsc

_NC = 2
_NS = 16
_NW = _NC * _NS
_CHUNK = 128
_WB = 448
_D = 16
_K = 2


def _head_body(x_ref, w_ref, o_ref):
    res = jnp.dot(x_ref[...], w_ref[...], preferred_element_type=jnp.float32)
    o_ref[0] = res[:, :_D]
    o_ref[1] = res[:, _D:]


def _head(x, w, block_rows=2000):
    n, k = x.shape
    return pl.pallas_call(
        _head_body,
        grid=(n // block_rows,),
        in_specs=[
            pl.BlockSpec((block_rows, k), lambda i: (i, 0)),
            pl.BlockSpec((k, 2 * _D), lambda i: (0, 0)),
        ],
        out_specs=pl.BlockSpec((2, block_rows, _D), lambda i: (0, i, 0)),
        out_shape=jax.ShapeDtypeStruct((2, n, _D), jnp.float32),
    )(x, w)


def _mid_body(lo0_ref, lo1_ref, hi0_ref, hi1_ref, y0_ref, y1_ref,
              b1l_ref, b1h_ref, w2l_ref, w2h_ref, b2_ref, w3_ref, z_ref):
    h_lo = jnp.maximum(y0_ref[0] + lo0_ref[0] + lo1_ref[0] + b1l_ref[...], 0.0)
    h_hi = jnp.maximum(y1_ref[0] + hi0_ref[0] + hi1_ref[0] + b1h_ref[...], 0.0)
    t = (jnp.dot(h_lo, w2l_ref[...], preferred_element_type=jnp.float32)
         + jnp.dot(h_hi, w2h_ref[...], preferred_element_type=jnp.float32))
    h = jnp.maximum(t + b2_ref[...], 0.0)
    z_ref[...] = jnp.dot(h, w3_ref[...], preferred_element_type=jnp.float32)


def _mid(parts_lo, parts_hi, y2, b1, w2, b2, w3, block_rows=2000):
    n = y2.shape[1]
    m = w3.shape[1]
    part_spec0 = pl.BlockSpec((1, block_rows, _D), lambda i: (0, i, 0))
    part_spec1 = pl.BlockSpec((1, block_rows, _D), lambda i: (1, i, 0))
    return pl.pallas_call(
        _mid_body,
        grid=(n // block_rows,),
        in_specs=[
            part_spec0, part_spec1, part_spec0, part_spec1,
            part_spec0, part_spec1,
            pl.BlockSpec((1, _D), lambda i: (0, 0)),
            pl.BlockSpec((1, _D), lambda i: (0, 0)),
            pl.BlockSpec((_D, 2 * _D), lambda i: (0, 0)),
            pl.BlockSpec((_D, 2 * _D), lambda i: (0, 0)),
            pl.BlockSpec((1, 2 * _D), lambda i: (0, 0)),
            pl.BlockSpec((2 * _D, m), lambda i: (0, 0)),
        ],
        out_specs=pl.BlockSpec((block_rows, m), lambda i: (i, 0)),
        out_shape=jax.ShapeDtypeStruct((n, m), jnp.float32),
    )(parts_lo, parts_lo, parts_hi, parts_hi, y2, y2,
      b1[:, :_D], b1[:, _D:], w2[:_D, :], w2[_D:, :], b2, w3)


def _out_body(q0_ref, q1_ref, z_ref, b3_ref, w4_ref, b4_ref, o_ref):
    h = z_ref[...] + q0_ref[0] + q1_ref[0] + b3_ref[...]
    h = jnp.maximum(h, 0.0)
    o_ref[...] = jnp.dot(h, w4_ref[...],
                         preferred_element_type=jnp.float32) + b4_ref[...]


def _final(parts, z, b3, w4, b4, block_rows=2000):
    n, d = z.shape
    m = w4.shape[1]
    return pl.pallas_call(
        _out_body,
        grid=(n // block_rows,),
        in_specs=[
            pl.BlockSpec((1, block_rows, d), lambda i: (0, i, 0)),
            pl.BlockSpec((1, block_rows, d), lambda i: (1, i, 0)),
            pl.BlockSpec((block_rows, d), lambda i: (i, 0)),
            pl.BlockSpec((1, d), lambda i: (0, 0)),
            pl.BlockSpec((d, m), lambda i: (0, 0)),
            pl.BlockSpec((1, m), lambda i: (0, 0)),
        ],
        out_specs=pl.BlockSpec((block_rows, m), lambda i: (i, 0)),
        out_shape=jax.ShapeDtypeStruct((n, m), jnp.float32),
    )(parts, parts, z, b3, w4, b4)


@functools.lru_cache(maxsize=None)
def _make_agg(n_acc, rows_per_tile, n_wb, c1):
    mesh = plsc.VectorSubcoreMesh(core_axis_name="c", subcore_axis_name="s")

    @functools.partial(
        pl.kernel,
        out_type=jax.ShapeDtypeStruct((_NC, n_acc, _D), jnp.float32),
        mesh=mesh,
        compiler_params=pltpu.CompilerParams(use_tc_tiling_on_sc=False),
        scratch_types=[
            pltpu.VMEM((c1, _CHUNK), jnp.int32),
            pltpu.VMEM((c1, _CHUNK), jnp.int32),
            pltpu.VMEM((4, _K, _CHUNK, _D), jnp.float32),
            pltpu.VMEM((_WB, _D), jnp.float32),
            pltpu.VMEM_SHARED((n_acc, _D), jnp.float32),
            pltpu.SemaphoreType.DMA((4,)),
            pltpu.SemaphoreType.DMA((4,)),
        ],
    )
    def agg(y_hbm, src_hbm, dst_hbm, out_hbm, src_v, dst_v, bufs_v, wb_v,
            acc_sh, gsem, ssem):
        c = lax.axis_index("c")
        s = lax.axis_index("s")
        wid = s * _NC + c
        base = s * rows_per_tile
        ng = c1 // _K

        zero16 = jnp.zeros((16,), jnp.float32)

        def zero_body(i, carry):
            wb_v[i, pl.ds(0, 16)] = zero16
            return carry

        idx_cp1 = pltpu.async_copy(src_hbm.at[wid], src_v, gsem.at[0])
        idx_cp2 = pltpu.async_copy(dst_hbm.at[wid], dst_v, gsem.at[1])

        lax.fori_loop(0, _WB, zero_body, 0)
        for k in range(n_wb):
            pltpu.sync_copy(wb_v, acc_sh.at[pl.ds(base + k * _WB, _WB)])

        idx_cp1.wait()
        idx_cp2.wait()
        plsc.subcore_barrier()

        def fire_gather(g, p):
            for b in range(_K):
                pltpu.async_copy(y_hbm.at[src_v.at[g * _K + b]],
                                 bufs_v.at[p, b], gsem.at[p])

        def drain_gather(g, p):
            for b in range(_K):
                pltpu.make_async_copy(y_hbm.at[src_v.at[g * _K + b]],
                                      bufs_v.at[p, b], gsem.at[p]).wait()

        def fire_scatter(g, p):
            for b in range(_K):
                pltpu.async_copy(bufs_v.at[p, b],
                                 acc_sh.at[dst_v.at[g * _K + b]],
                                 ssem.at[p], add=True)

        def drain_scatter(g, p):
            for b in range(_K):
                pltpu.make_async_copy(bufs_v.at[p, b],
                                      acc_sh.at[dst_v.at[g * _K + b]],
                                      ssem.at[p]).wait()

        fire_gather(0, 0)
        fire_gather(1, 1)

        def block(g, carry):
            p = lax.rem(g, 4)
            p2 = lax.rem(g + 2, 4)
            drain_gather(g, p)
            fire_scatter(g, p)

            @pl.when(g >= 2)
            def _():
                drain_scatter(g - 2, p2)

            @pl.when(g + 2 < ng)
            def _():
                fire_gather(g + 2, p2)

            return carry

        lax.fori_loop(0, ng, block, 0)
        drain_scatter(ng - 2, (ng - 2) % 4)
        drain_scatter(ng - 1, (ng - 1) % 4)
        plsc.subcore_barrier()

        pltpu.sync_copy(acc_sh.at[pl.ds(base, rows_per_tile)],
                        out_hbm.at[c, pl.ds(base, rows_per_tile)])

    return agg


def kernel(x, edge_index, w1, b1, w2, b2, w3, b3, w4, b4):
    n = x.shape[0]
    e = edge_index.shape[1]
    ei = edge_index.astype(jnp.int32)

    c1 = -(-e // (_NW * _CHUNK))
    c1 = -(-c1 // _K) * _K
    e_pad = _NW * c1 * _CHUNK
    rows_per_tile = -(-n // (_NS * _WB)) * _WB
    n_wb = rows_per_tile // _WB
    n_acc = _NS * rows_per_tile

    src = jnp.concatenate(
        [ei[0], jnp.zeros((e_pad - e,), jnp.int32)]).reshape(_NW, c1, _CHUNK)
    dst = jnp.concatenate(
        [ei[1], jnp.full((e_pad - e,), n, jnp.int32)]).reshape(_NW, c1, _CHUNK)

    agg = _make_agg(n_acc, rows_per_tile, n_wb, c1)

    y2 = _head(x, w1)
    parts_lo = agg(y2[0], src, dst)
    parts_hi = agg(y2[1], src, dst)
    z = _mid(parts_lo, parts_hi, y2, b1.reshape(1, -1), w2,
             b2.reshape(1, -1), w3)
    parts2 = agg(z, src, dst)
    return _final(parts2, z, b3.reshape(1, -1), w4, b4.reshape(1, -1))
